# baseline (speedup 1.0000x reference)
import jax
import jax.numpy as jnp
from jax.experimental import pallas as pl


def _add_body(x0_ref, x1_ref, pe_ref, out_ref):
    h2 = x0_ref.shape[-1]
    out_ref[0, :, :h2] = x0_ref[0] + pe_ref[:, :h2]
    out_ref[0, :, h2:] = x1_ref[0] + pe_ref[:, h2:]


def kernel(x, pe_table):
    B, S, H = x.shape
    BS = 2048
    H2 = H // 2
    grid = (S // BS, B)
    return pl.pallas_call(
        _add_body,
        grid=grid,
        in_specs=[
            pl.BlockSpec((1, BS, H2), lambda s, b: (b, s, 0)),
            pl.BlockSpec((1, BS, H2), lambda s, b: (b, s, 1)),
            pl.BlockSpec((BS, H), lambda s, b: (s, 0)),
        ],
        out_specs=pl.BlockSpec((1, BS, H), lambda s, b: (b, s, 0)),
        out_shape=jax.ShapeDtypeStruct((B, S, H), x.dtype),
    )(x, x, pe_table)


# speedup vs baseline: 1.0101x; 1.0101x over previous
import jax
import jax.numpy as jnp
from jax.experimental import pallas as pl
from jax.experimental.pallas import tpu as pltpu

_NBUF = 3
_BS = 1024


def _body(x_hbm, pe_hbm, out_hbm, pe_vmem, xbuf, obuf, xsem, osem, pesem):
    n_rows = x_hbm.shape[0]
    pe_rows = pe_hbm.shape[0]
    n_blocks = n_rows // _BS

    pltpu.make_async_copy(pe_hbm, pe_vmem, pesem).start()
    for k in range(_NBUF):
        pltpu.make_async_copy(
            x_hbm.at[pl.ds(k * _BS, _BS), :], xbuf.at[k], xsem.at[k]
        ).start()
    pltpu.make_async_copy(pe_hbm, pe_vmem, pesem).wait()

    def step(i, _):
        k = jax.lax.rem(i, _NBUF)
        row0 = i * _BS
        pe0 = jax.lax.rem(row0, pe_rows)
        pltpu.make_async_copy(
            x_hbm.at[pl.ds(row0, _BS), :], xbuf.at[k], xsem.at[k]
        ).wait()

        @pl.when(i >= _NBUF)
        def _():
            prev0 = (i - _NBUF) * _BS
            pltpu.make_async_copy(
                obuf.at[k], out_hbm.at[pl.ds(prev0, _BS), :], osem.at[k]
            ).wait()

        obuf[k] = xbuf[k] + pe_vmem[pl.ds(pe0, _BS), :]
        pltpu.make_async_copy(
            obuf.at[k], out_hbm.at[pl.ds(row0, _BS), :], osem.at[k]
        ).start()

        @pl.when(i + _NBUF < n_blocks)
        def _():
            nxt0 = (i + _NBUF) * _BS
            pltpu.make_async_copy(
                x_hbm.at[pl.ds(nxt0, _BS), :], xbuf.at[k], xsem.at[k]
            ).start()

        return 0

    jax.lax.fori_loop(0, n_blocks, step, 0)

    for k in range(_NBUF):
        i = n_blocks - _NBUF + k
        pltpu.make_async_copy(
            obuf.at[i % _NBUF], out_hbm.at[pl.ds(i * _BS, _BS), :], osem.at[i % _NBUF]
        ).wait()


def kernel(x, pe_table):
    B, S, H = x.shape
    x2d = x.reshape(B * S, H)
    out2d = pl.pallas_call(
        _body,
        in_specs=[
            pl.BlockSpec(memory_space=pltpu.MemorySpace.HBM),
            pl.BlockSpec(memory_space=pltpu.MemorySpace.HBM),
        ],
        out_specs=pl.BlockSpec(memory_space=pltpu.MemorySpace.HBM),
        out_shape=jax.ShapeDtypeStruct((B * S, H), x.dtype),
        scratch_shapes=[
            pltpu.VMEM((S, H), x.dtype),
            pltpu.VMEM((_NBUF, _BS, H), x.dtype),
            pltpu.VMEM((_NBUF, _BS, H), x.dtype),
            pltpu.SemaphoreType.DMA((_NBUF,)),
            pltpu.SemaphoreType.DMA((_NBUF,)),
            pltpu.SemaphoreType.DMA,
        ],
    )(x2d, pe_table)
    return out2d.reshape(B, S, H)
